# baseline (speedup 1.0000x reference)
import functools

import jax
import jax.numpy as jnp
from jax import lax
from jax.experimental import pallas as pl
from jax.experimental.pallas import tpu as pltpu
from jax.experimental.pallas import tpu_sc as plsc

N = 10000
R = 4
E = 80000
D = 128
HD = D // 2
K = 128
NS = 16
NC = 2
E2 = 81920
PT = E2 // NS
ITERS = PT // K
BROWS = E2 // K
NROW = R * 2 * BROWS
NTOT = N + 8
DEGW = 16
NB = 3
CH = 40
MAXCH = 16


def _sc_body(x_hbm, ei_hbm, agg_out, deg_out,
             srcbig, dstbig, srcb0, srcb1, srcb2, dstb0, dstb1, dstb2,
             rows0, rows1, rows2, ones, zrow, zdeg,
             aggb, degb, agg_sh, deg_sh,
             sg0, sg1, sg2, sd0, sd1, sd2):
    srcb = [srcb0, srcb1, srcb2]
    dstb = [dstb0, dstb1, dstb2]
    rows = [rows0, rows1, rows2]
    sg = [sg0, sg1, sg2]
    sd = [sd0, sd1, sd2]

    c = lax.axis_index("c")
    s = lax.axis_index("s")

    def fill_ones(i, carry):
        ones[i] = jnp.full((16,), 1.0, jnp.float32)
        return carry
    lax.fori_loop(0, K, fill_ones, 0)

    def fill_zrow(i, carry):
        for jj in range(HD // 16):
            zrow[i, pl.ds(jj * 16, 16)] = jnp.zeros((16,), jnp.float32)
        return carry
    lax.fori_loop(0, CH, fill_zrow, 0)

    def fill_zdeg(i, carry):
        zdeg[i] = jnp.zeros((16,), jnp.float32)
        return carry
    lax.fori_loop(0, CH, fill_zdeg, 0)

    row0 = jnp.where(s < 10, 640 * s, 6400 + 600 * (s - 10))
    nch = jnp.where(s < 10, 16, 15)
    xoff = c * N

    for r in range(R):
        def zero_body(z, carry):
            @pl.when(z < nch)
            def _():
                sl = pl.ds(row0 + z * CH, CH)
                pltpu.sync_copy(zrow, agg_sh.at[sl, :])

                @pl.when(c == 0)
                def _():
                    pltpu.sync_copy(zdeg, deg_sh.at[sl, :])
            return carry
        lax.fori_loop(0, MAXCH, zero_body, 0)

        srow = pl.multiple_of((2 * r) * BROWS + s * ITERS, 8)
        drow = pl.multiple_of((2 * r + 1) * BROWS + s * ITERS, 8)
        pltpu.sync_copy(ei_hbm.at[pl.ds(srow, ITERS), :], srcbig)
        pltpu.sync_copy(ei_hbm.at[pl.ds(drow, ITERS), :], dstbig)
        plsc.subcore_barrier()

        def edge_body(i, carry):
            for b in range(NB):
                @pl.when((i < ITERS) & (i % NB == b))
                def _(b=b):
                    @pl.when((i >= NB) & (c == 0))
                    def _():
                        pltpu.make_async_copy(
                            ones, deg_sh.at[dstb[b]], sd[b]).wait()

                    def cp(jj, carry2):
                        sl = pl.ds(jj * 16, 16)
                        srcb[b][sl] = srcbig[i, sl] + xoff
                        dstb[b][sl] = dstbig[i, sl]
                        return carry2
                    lax.fori_loop(0, K // 16, cp, 0)
                    pltpu.async_copy(x_hbm.at[srcb[b]], rows[b], sg[b])
            for b in range(NB):
                @pl.when((i >= 1) & ((i - 1) % NB == b))
                def _(b=b):
                    pltpu.make_async_copy(
                        x_hbm.at[srcb[b]], rows[b], sg[b]).wait()
                    pltpu.sync_copy(rows[b], agg_sh.at[dstb[b]], add=True)

                    @pl.when(c == 0)
                    def _():
                        pltpu.async_copy(
                            ones, deg_sh.at[dstb[b]], sd[b], add=True)
            return carry
        lax.fori_loop(0, ITERS + 1, edge_body, 0)
        for b in range(NB):
            @pl.when(c == 0)
            def _(b=b):
                pltpu.make_async_copy(
                    ones, deg_sh.at[dstb[b]], sd[b]).wait()
        plsc.subcore_barrier()

        def out_body(z, carry):
            @pl.when(z < nch)
            def _():
                sl = pl.ds(row0 + z * CH, CH)
                pltpu.sync_copy(agg_sh.at[sl, :], aggb)
                pltpu.sync_copy(aggb, agg_out.at[c, r, sl, :])

                @pl.when(c == 0)
                def _():
                    pltpu.sync_copy(deg_sh.at[sl, :], degb)
                    pltpu.sync_copy(degb, deg_out.at[r, sl, :])
            return carry
        lax.fori_loop(0, MAXCH, out_body, 0)

        plsc.subcore_barrier()


_sc_aggregate = functools.partial(
    pl.kernel,
    out_type=[
        jax.ShapeDtypeStruct((NC, R, N, HD), jnp.float32),
        jax.ShapeDtypeStruct((R, N, DEGW), jnp.float32),
    ],
    mesh=plsc.VectorSubcoreMesh(core_axis_name="c", subcore_axis_name="s"),
    compiler_params=pltpu.CompilerParams(use_tc_tiling_on_sc=False),
    scratch_types=(
        [
            pltpu.VMEM((ITERS, K), jnp.int32),
            pltpu.VMEM((ITERS, K), jnp.int32),
        ]
        + [pltpu.VMEM((K,), jnp.int32) for _ in range(2 * NB)]
        + [pltpu.VMEM((K, HD), jnp.float32) for _ in range(NB)]
        + [
            pltpu.VMEM((K, DEGW), jnp.float32),
            pltpu.VMEM((CH, HD), jnp.float32),
            pltpu.VMEM((CH, DEGW), jnp.float32),
            pltpu.VMEM((CH, HD), jnp.float32),
            pltpu.VMEM((CH, DEGW), jnp.float32),
            pltpu.VMEM_SHARED((NTOT, HD), jnp.float32),
            pltpu.VMEM_SHARED((NTOT, DEGW), jnp.float32),
        ]
        + [pltpu.SemaphoreType.DMA for _ in range(2 * NB)]
    ),
)(_sc_body)


BN = 1000


def _tc_body(agg_ref, deg_ref, w_ref, b_ref, o_ref):
    acc = jnp.zeros((BN, D), jnp.float32)
    for r in range(R):
        deg = deg_ref[r, :, 0:1]
        norm = 1.0 / jnp.maximum(deg, 1.0)
        a = jnp.concatenate([agg_ref[0, r], agg_ref[1, r]], axis=-1)
        acc = acc + jnp.dot(a * norm, w_ref[r],
                            preferred_element_type=jnp.float32)
    o_ref[...] = acc + b_ref[...]


def _tc_combine(agg, deg, weight, bias):
    return pl.pallas_call(
        _tc_body,
        grid=(N // BN,),
        in_specs=[
            pl.BlockSpec((NC, R, BN, HD), lambda i: (0, 0, i, 0)),
            pl.BlockSpec((R, BN, DEGW), lambda i: (0, i, 0)),
            pl.BlockSpec((R, D, D), lambda i: (0, 0, 0)),
            pl.BlockSpec((1, D), lambda i: (0, 0)),
        ],
        out_specs=pl.BlockSpec((BN, D), lambda i: (i, 0)),
        out_shape=jax.ShapeDtypeStruct((N, D), jnp.float32),
    )(agg, deg, weight, bias)


def kernel(x, edge_index_0, edge_index_1, edge_index_2, edge_index_3,
           weight, h_bias):
    xsplit = jnp.concatenate([x[:, :HD], x[:, HD:]], axis=0)
    pad_s = jnp.zeros((E2 - E,), jnp.int32)
    pad_d = jnp.full((E2 - E,), N, jnp.int32)
    parts = []
    for ei in (edge_index_0, edge_index_1, edge_index_2, edge_index_3):
        parts.append(jnp.concatenate([ei[0], pad_s]))
        parts.append(jnp.concatenate([ei[1], pad_d]))
    eib = jnp.concatenate(parts).reshape(NROW, K)
    agg, deg = _sc_aggregate(xsplit, eib)
    return _tc_combine(agg, deg, weight, h_bias.reshape(1, D))


# speedup vs baseline: 1.0152x; 1.0152x over previous
import functools

import jax
import jax.numpy as jnp
from jax import lax
from jax.experimental import pallas as pl
from jax.experimental.pallas import tpu as pltpu
from jax.experimental.pallas import tpu_sc as plsc

N = 10000
R = 4
E = 80000
D = 128
HD = D // 2
K = 128
NS = 16
NC = 2
E2 = 81920
PT = E2 // NS
ITERS = PT // K
BROWS = E2 // K
NROW = R * 2 * BROWS
NTOT = N + 8
DEGW = 16
NB = 3
CH = 40
MAXCH = 16


def _sc_body(x_hbm, ei_hbm, agg_out, deg_out,
             srcbig, dstbig, srcb0, srcb1, srcb2, dstb0, dstb1, dstb2,
             rows0, rows1, rows2, ones, zrow, zdeg,
             aggb, degb, agg_sh, deg_sh,
             sg0, sg1, sg2, sd0, sd1, sd2, sa0, sa1, sa2):
    srcb = [srcb0, srcb1, srcb2]
    dstb = [dstb0, dstb1, dstb2]
    rows = [rows0, rows1, rows2]
    sg = [sg0, sg1, sg2]
    sd = [sd0, sd1, sd2]
    sa = [sa0, sa1, sa2]

    c = lax.axis_index("c")
    s = lax.axis_index("s")

    def fill_ones(i, carry):
        ones[i] = jnp.full((16,), 1.0, jnp.float32)
        return carry
    lax.fori_loop(0, K, fill_ones, 0)

    def fill_zrow(i, carry):
        for jj in range(HD // 16):
            zrow[i, pl.ds(jj * 16, 16)] = jnp.zeros((16,), jnp.float32)
        return carry
    lax.fori_loop(0, CH, fill_zrow, 0)

    def fill_zdeg(i, carry):
        zdeg[i] = jnp.zeros((16,), jnp.float32)
        return carry
    lax.fori_loop(0, CH, fill_zdeg, 0)

    row0 = jnp.where(s < 10, 640 * s, 6400 + 600 * (s - 10))
    nch = jnp.where(s < 10, 16, 15)
    xoff = c * N

    for r in range(R):
        def zero_body(z, carry):
            @pl.when(z < nch)
            def _():
                sl = pl.ds(row0 + z * CH, CH)
                pltpu.sync_copy(zrow, agg_sh.at[sl, :])

                @pl.when(c == 0)
                def _():
                    pltpu.sync_copy(zdeg, deg_sh.at[sl, :])
            return carry
        lax.fori_loop(0, MAXCH, zero_body, 0)

        srow = pl.multiple_of((2 * r) * BROWS + s * ITERS, 8)
        drow = pl.multiple_of((2 * r + 1) * BROWS + s * ITERS, 8)
        pltpu.sync_copy(ei_hbm.at[pl.ds(srow, ITERS), :], srcbig)
        pltpu.sync_copy(ei_hbm.at[pl.ds(drow, ITERS), :], dstbig)
        plsc.subcore_barrier()

        def edge_body(i, carry):
            for b in range(NB):
                @pl.when((i < ITERS) & (i % NB == b))
                def _(b=b):
                    @pl.when(i >= NB)
                    def _():
                        pltpu.make_async_copy(
                            rows[b], agg_sh.at[dstb[b]], sa[b]).wait()

                        @pl.when(c == 0)
                        def _():
                            pltpu.make_async_copy(
                                ones, deg_sh.at[dstb[b]], sd[b]).wait()

                    for jj in range(K // 16):
                        sl = pl.ds(jj * 16, 16)
                        srcb[b][sl] = srcbig[i, sl] + xoff
                        dstb[b][sl] = dstbig[i, sl]
                    pltpu.async_copy(x_hbm.at[srcb[b]], rows[b], sg[b])
            for b in range(NB):
                @pl.when((i >= 2) & (i < ITERS + 2) & ((i - 2) % NB == b))
                def _(b=b):
                    pltpu.make_async_copy(
                        x_hbm.at[srcb[b]], rows[b], sg[b]).wait()
                    pltpu.async_copy(
                        rows[b], agg_sh.at[dstb[b]], sa[b], add=True)

                    @pl.when(c == 0)
                    def _():
                        pltpu.async_copy(
                            ones, deg_sh.at[dstb[b]], sd[b], add=True)
            return carry
        lax.fori_loop(0, ITERS + 2, edge_body, 0)
        for b in range(NB):
            pltpu.make_async_copy(
                rows[b], agg_sh.at[dstb[b]], sa[b]).wait()

            @pl.when(c == 0)
            def _(b=b):
                pltpu.make_async_copy(
                    ones, deg_sh.at[dstb[b]], sd[b]).wait()
        plsc.subcore_barrier()

        def out_body(z, carry):
            @pl.when(z < nch)
            def _():
                sl = pl.ds(row0 + z * CH, CH)
                pltpu.sync_copy(agg_sh.at[sl, :], aggb)
                pltpu.sync_copy(aggb, agg_out.at[c, r, sl, :])

                @pl.when(c == 0)
                def _():
                    pltpu.sync_copy(deg_sh.at[sl, :], degb)
                    pltpu.sync_copy(degb, deg_out.at[r, sl, :])
            return carry
        lax.fori_loop(0, MAXCH, out_body, 0)

        plsc.subcore_barrier()


_sc_aggregate = functools.partial(
    pl.kernel,
    out_type=[
        jax.ShapeDtypeStruct((NC, R, N, HD), jnp.float32),
        jax.ShapeDtypeStruct((R, N, DEGW), jnp.float32),
    ],
    mesh=plsc.VectorSubcoreMesh(core_axis_name="c", subcore_axis_name="s"),
    compiler_params=pltpu.CompilerParams(use_tc_tiling_on_sc=False),
    scratch_types=(
        [
            pltpu.VMEM((ITERS, K), jnp.int32),
            pltpu.VMEM((ITERS, K), jnp.int32),
        ]
        + [pltpu.VMEM((K,), jnp.int32) for _ in range(2 * NB)]
        + [pltpu.VMEM((K, HD), jnp.float32) for _ in range(NB)]
        + [
            pltpu.VMEM((K, DEGW), jnp.float32),
            pltpu.VMEM((CH, HD), jnp.float32),
            pltpu.VMEM((CH, DEGW), jnp.float32),
            pltpu.VMEM((CH, HD), jnp.float32),
            pltpu.VMEM((CH, DEGW), jnp.float32),
            pltpu.VMEM_SHARED((NTOT, HD), jnp.float32),
            pltpu.VMEM_SHARED((NTOT, DEGW), jnp.float32),
        ]
        + [pltpu.SemaphoreType.DMA for _ in range(3 * NB)]
    ),
)(_sc_body)


BN = 1000


def _tc_body(agg_ref, deg_ref, w_ref, b_ref, o_ref):
    acc = jnp.zeros((BN, D), jnp.float32)
    for r in range(R):
        deg = deg_ref[r, :, 0:1]
        norm = 1.0 / jnp.maximum(deg, 1.0)
        a = jnp.concatenate([agg_ref[0, r], agg_ref[1, r]], axis=-1)
        acc = acc + jnp.dot(a * norm, w_ref[r],
                            preferred_element_type=jnp.float32)
    o_ref[...] = acc + b_ref[...]


def _tc_combine(agg, deg, weight, bias):
    return pl.pallas_call(
        _tc_body,
        grid=(N // BN,),
        in_specs=[
            pl.BlockSpec((NC, R, BN, HD), lambda i: (0, 0, i, 0)),
            pl.BlockSpec((R, BN, DEGW), lambda i: (0, i, 0)),
            pl.BlockSpec((R, D, D), lambda i: (0, 0, 0)),
            pl.BlockSpec((1, D), lambda i: (0, 0)),
        ],
        out_specs=pl.BlockSpec((BN, D), lambda i: (i, 0)),
        out_shape=jax.ShapeDtypeStruct((N, D), jnp.float32),
    )(agg, deg, weight, bias)


def kernel(x, edge_index_0, edge_index_1, edge_index_2, edge_index_3,
           weight, h_bias):
    xsplit = jnp.concatenate([x[:, :HD], x[:, HD:]], axis=0)
    pad_s = jnp.zeros((E2 - E,), jnp.int32)
    pad_d = jnp.full((E2 - E,), N, jnp.int32)
    parts = []
    for ei in (edge_index_0, edge_index_1, edge_index_2, edge_index_3):
        parts.append(jnp.concatenate([ei[0], pad_s]))
        parts.append(jnp.concatenate([ei[1], pad_d]))
    eib = jnp.concatenate(parts).reshape(NROW, K)
    agg, deg = _sc_aggregate(xsplit, eib)
    return _tc_combine(agg, deg, weight, h_bias.reshape(1, D))
